# baseline (speedup 1.0000x reference)
import functools

import jax
import jax.numpy as jnp
from jax import lax
from jax.experimental import pallas as pl
from jax.experimental.pallas import tpu as pltpu
from jax.experimental.pallas import tpu_sc as plsc

_B = 4096
_L = 200
_CTX = 32
_NW = 32
_UR = 1024
_NU2 = _L * (_B // _UR)
_NU = _NU2 + _B // _UR
_UPW = -(-_NU // _NW)
_PLANE = _CTX * _B
_RCHUNK = 8 * 8 * 128
_O2_LEN = _L * _PLANE
_O1_LEN = _PLANE


def _tanh16(x):
    e = jnp.exp(x + x)
    return 1.0 - 2.0 / (e + 1.0)


def _body(topics_hbm, absidx_hbm, table_hbm, out1_hbm, out2_hbm,
          idx_v, g_v, ub_v, sem):
    wid = lax.axis_index("s") * 2 + lax.axis_index("c")
    lane = lax.iota(jnp.int32, 16)
    lanepat = ((lane >> 3) << 13) + ((lane & 7) << 7)

    def unit_iter(k, carry):
        u = wid + _NW * k

        @pl.when(u < _NU)
        def _run_unit():
            @pl.when(u < _NU2)
            def _():
                pltpu.sync_copy(absidx_hbm.at[pl.ds(u * _UR, _UR)], idx_v)

            @pl.when(u >= _NU2)
            def _():
                pltpu.sync_copy(
                    topics_hbm.at[pl.ds((u - _NU2) * _UR, _UR)], idx_v)

            pltpu.async_copy(table_hbm.at[idx_v], g_v, sem).wait()

            def row_iter(j, c):
                b0 = j * 4
                blk = b0 >> 7
                base0 = (blk << 10) + (b0 & 127)
                for t in range(4):
                    b = b0 + t
                    idxlo = lanepat + (base0 + t)
                    x0 = g_v[b, pl.ds(0, 16)]
                    x1 = g_v[b, pl.ds(16, 16)]
                    plsc.store_scatter(ub_v, [idxlo], _tanh16(x0))
                    plsc.store_scatter(ub_v, [idxlo + 16384], _tanh16(x1))
                return c

            lax.fori_loop(0, _UR // 4, row_iter, 0)

            q = jnp.where(u < _NU2, u & 3, u - _NU2)
            l = u >> 2

            @pl.when(u < _NU2)
            def _():
                for r in range(4):
                    pltpu.sync_copy(
                        ub_v.at[pl.ds(r * _RCHUNK, _RCHUNK)],
                        out2_hbm.at[pl.ds(
                            l * _PLANE + r * (8 * _B) + q * _RCHUNK,
                            _RCHUNK)])

            @pl.when(u >= _NU2)
            def _():
                for r in range(4):
                    pltpu.sync_copy(
                        ub_v.at[pl.ds(r * _RCHUNK, _RCHUNK)],
                        out1_hbm.at[pl.ds(r * (8 * _B) + q * _RCHUNK,
                                          _RCHUNK)])

        return carry

    lax.fori_loop(0, _UPW, unit_iter, 0)


_mesh = plsc.VectorSubcoreMesh(core_axis_name="c", subcore_axis_name="s")

_gather_tanh = functools.partial(
    pl.kernel,
    out_type=(
        jax.ShapeDtypeStruct((_O1_LEN,), jnp.float32),
        jax.ShapeDtypeStruct((_O2_LEN,), jnp.float32),
    ),
    mesh=_mesh,
    scratch_types=[
        pltpu.VMEM((_UR,), jnp.int32),
        pltpu.VMEM((_UR, _CTX), jnp.float32),
        pltpu.VMEM((4 * _RCHUNK,), jnp.float32),
        pltpu.SemaphoreType.DMA,
    ],
    compiler_params=pltpu.CompilerParams(
        use_tc_tiling_on_sc=False, needs_layout_passes=False),
)(_body)


def kernel(topics, structure_abstracts, embedding):
    absidx = structure_abstracts.T.reshape(-1).astype(jnp.int32)
    o1, o2 = _gather_tanh(topics.astype(jnp.int32), absidx, embedding)
    out1 = (o1.reshape(4, 32, 8, 128)
            .transpose(1, 3, 0, 2).reshape(_B, 1, _CTX))
    out2 = (o2.reshape(_L, 4, 32, 8, 128)
            .transpose(2, 4, 0, 1, 3).reshape(_B, _L, _CTX))
    return (out1, out2)


# speedup vs baseline: 1.5217x; 1.5217x over previous
import functools

import jax
import jax.numpy as jnp
from jax import lax
from jax.experimental import pallas as pl
from jax.experimental.pallas import tpu as pltpu
from jax.experimental.pallas import tpu_sc as plsc

_B = 4096
_L = 200
_CTX = 32
_NW = 32
_UR = 1024
_NU2 = _L * _B // _UR
_NU = _NU2 + _B // _UR
_UPW = -(-_NU // _NW)


def _tanh16(x):
    e = jnp.exp(x + x)
    return 1.0 - 2.0 / (e + 1.0)


def _body(topics_hbm, absidx_hbm, table_hbm, out1_hbm, out2_hbm,
          idx_v, g_v, sem0, sem1, osem0, osem1):
    wid = lax.axis_index("s") * 2 + lax.axis_index("c")
    gsems = (sem0, sem1)
    osems = (osem0, osem1)

    def load_idx(k, buf):
        u = wid + _NW * k

        @pl.when(u < _NU2)
        def _():
            pltpu.sync_copy(absidx_hbm.at[pl.ds(u * _UR, _UR)],
                            idx_v.at[buf])

        @pl.when(jnp.logical_and(u >= _NU2, u < _NU))
        def _():
            pltpu.sync_copy(topics_hbm.at[pl.ds((u - _NU2) * _UR, _UR)],
                            idx_v.at[buf])

    def start_gather(k, buf):
        @pl.when(wid + _NW * k < _NU)
        def _():
            pltpu.async_copy(table_hbm.at[idx_v.at[buf]], g_v.at[buf],
                             gsems[buf])

    def wait_gather(buf):
        pltpu.make_async_copy(table_hbm.at[idx_v.at[buf]], g_v.at[buf],
                              gsems[buf]).wait()

    def write_out(k, buf):
        u = wid + _NW * k

        @pl.when(u < _NU2)
        def _():
            pltpu.async_copy(g_v.at[buf],
                             out2_hbm.at[pl.ds(u * _UR, _UR)], osems[buf])

        @pl.when(jnp.logical_and(u >= _NU2, u < _NU))
        def _():
            pltpu.async_copy(g_v.at[buf],
                             out1_hbm.at[pl.ds((u - _NU2) * _UR, _UR)],
                             osems[buf])

    def wait_out(k, buf):
        u = wid + _NW * k

        @pl.when(u < _NU2)
        def _():
            pltpu.make_async_copy(
                g_v.at[buf], out2_hbm.at[pl.ds(u * _UR, _UR)],
                osems[buf]).wait()

        @pl.when(jnp.logical_and(u >= _NU2, u < _NU))
        def _():
            pltpu.make_async_copy(
                g_v.at[buf], out1_hbm.at[pl.ds((u - _NU2) * _UR, _UR)],
                osems[buf]).wait()

    load_idx(0, 0)
    start_gather(0, 0)

    def unit_pair_iter(k2, carry):
        for b in range(2):
            k = 2 * k2 + b
            nb = 1 - b
            u = wid + _NW * k

            @pl.when(k >= 1)
            def _(k=k, nb=nb):
                wait_out(k - 1, nb)

            @pl.when(u + _NW < _NU)
            def _(k=k, nb=nb):
                load_idx(k + 1, nb)

            start_gather(k + 1, nb)

            @pl.when(u < _NU)
            def _run_unit(k=k, b=b):
                wait_gather(b)

                def row_iter(j, c):
                    r0 = j * 4
                    for t in range(4):
                        for h in range(2):
                            sl = (b, r0 + t, pl.ds(16 * h, 16))
                            g_v[sl] = _tanh16(g_v[sl])
                    return c

                lax.fori_loop(0, _UR // 4, row_iter, 0)
                write_out(k, b)

        return carry

    lax.fori_loop(0, _UPW // 2, unit_pair_iter, 0)
    wait_out(_UPW - 1, (_UPW - 1) % 2)


_mesh = plsc.VectorSubcoreMesh(core_axis_name="c", subcore_axis_name="s")

_gather_tanh = functools.partial(
    pl.kernel,
    out_type=(
        jax.ShapeDtypeStruct((_B, _CTX), jnp.float32),
        jax.ShapeDtypeStruct((_L * _B, _CTX), jnp.float32),
    ),
    mesh=_mesh,
    scratch_types=[
        pltpu.VMEM((2, _UR), jnp.int32),
        pltpu.VMEM((2, _UR, _CTX), jnp.float32),
        pltpu.SemaphoreType.DMA,
        pltpu.SemaphoreType.DMA,
        pltpu.SemaphoreType.DMA,
        pltpu.SemaphoreType.DMA,
    ],
    compiler_params=pltpu.CompilerParams(
        use_tc_tiling_on_sc=False, needs_layout_passes=False),
)(_body)


def kernel(topics, structure_abstracts, embedding):
    absidx = structure_abstracts.T.reshape(-1).astype(jnp.int32)
    o1, o2 = _gather_tanh(topics.astype(jnp.int32), absidx, embedding)
    out1 = o1.reshape(_B, 1, _CTX)
    out2 = o2.reshape(_L, _B, _CTX).transpose(1, 0, 2)
    return (out1, out2)
